# Initial kernel scaffold; baseline (speedup 1.0000x reference)
import jax
import jax.numpy as jnp
from jax.experimental import pallas as pl


def kernel(x, W, edge_index, adj_vals):
    raise NotImplementedError("write your pallas kernel here")


# speedup vs baseline: 4.4922x; 4.4922x over previous
import functools

import jax
import jax.numpy as jnp
from jax import lax
from jax.experimental import pallas as pl
from jax.experimental.pallas import tpu as pltpu
from jax.experimental.pallas import tpu_sc as plsc

N = 10000
E = 320000
D = 128

NC = 2
NS = 16
NW = NC * NS
LANES = 16

CHUNK = 128
CHUNKS_PER_TILE = -(-E // (NW * CHUNK))
EDGES_PER_TILE = CHUNKS_PER_TILE * CHUNK
E_PAD = EDGES_PER_TILE * NW
N_PAD = 10240
ROWS_PER_TILE = N_PAD // NS


def _matmul_body(x_ref, w_ref, o_ref):
    o_ref[...] = jnp.dot(x_ref[...], w_ref[...], preferred_element_type=jnp.float32)


def _matmul(x, W):
    bm = 1000
    return pl.pallas_call(
        _matmul_body,
        grid=(N // bm,),
        in_specs=[
            pl.BlockSpec((bm, D), lambda i: (i, 0)),
            pl.BlockSpec((D, D), lambda i: (0, 0)),
        ],
        out_specs=pl.BlockSpec((bm, D), lambda i: (i, 0)),
        out_shape=jax.ShapeDtypeStruct((N, D), jnp.float32),
    )(x, W)


def _combine_body(a_ref, b_ref, o_ref):
    o_ref[...] = jnp.maximum(a_ref[...] + b_ref[...], 0.0)


def _combine(a, b):
    bm = 1000
    return pl.pallas_call(
        _combine_body,
        grid=(N // bm,),
        in_specs=[
            pl.BlockSpec((bm, D), lambda i: (i, 0)),
            pl.BlockSpec((bm, D), lambda i: (i, 0)),
        ],
        out_specs=pl.BlockSpec((bm, D), lambda i: (i, 0)),
        out_shape=jax.ShapeDtypeStruct((N, D), jnp.float32),
    )(a, b)


_mesh = plsc.VectorSubcoreMesh(core_axis_name="c", subcore_axis_name="s")


@functools.partial(
    pl.kernel,
    mesh=_mesh,
    out_type=jax.ShapeDtypeStruct((NC, N_PAD, D), jnp.float32),
    scratch_types=[
        pltpu.VMEM((CHUNKS_PER_TILE, CHUNK), jnp.int32),
        pltpu.VMEM((CHUNKS_PER_TILE, CHUNK), jnp.int32),
        pltpu.VMEM((CHUNKS_PER_TILE, CHUNK), jnp.float32),
        pltpu.VMEM((CHUNK, D), jnp.float32),
        pltpu.VMEM_SHARED((N_PAD, D), jnp.float32),
        pltpu.SemaphoreType.DMA,
    ],
)
def _sc_edge_kernel(h_hbm, src_hbm, dst_hbm, val_hbm, part_hbm,
                    src_v, dst_v, val_v, rows_v, acc_sh, sem):
    cid = lax.axis_index("c")
    sid = lax.axis_index("s")
    wid = sid * NC + cid

    pltpu.sync_copy(src_hbm.at[wid], src_v)
    pltpu.sync_copy(dst_hbm.at[wid], dst_v)
    pltpu.sync_copy(val_hbm.at[wid], val_v)

    def _zero_row(r, carry):
        for j in range(D // LANES):
            rows_v[r, pl.ds(j * LANES, LANES)] = jnp.zeros((LANES,), jnp.float32)
        return carry

    lax.fori_loop(0, CHUNK, _zero_row, 0)

    base = sid * ROWS_PER_TILE

    def _zero_acc(b, carry):
        pltpu.sync_copy(rows_v, acc_sh.at[pl.ds(base + b * CHUNK, CHUNK)])
        return carry

    lax.fori_loop(0, ROWS_PER_TILE // CHUNK, _zero_acc, 0)
    rem = ROWS_PER_TILE % CHUNK
    if rem:
        pltpu.sync_copy(
            rows_v.at[pl.ds(0, rem)],
            acc_sh.at[pl.ds(base + (ROWS_PER_TILE // CHUNK) * CHUNK, rem)],
        )

    plsc.subcore_barrier()

    def _chunk(c, carry):
        pltpu.async_copy(h_hbm.at[src_v.at[c]], rows_v, sem).wait()

        def _group(g, inner):
            vvec = val_v[c, pl.ds(g * LANES, LANES)]
            for i in range(LANES):
                v = vvec[i]
                e = g * LANES + i
                for j in range(D // LANES):
                    sl = pl.ds(j * LANES, LANES)
                    rows_v[e, sl] = rows_v[e, sl] * v
            return inner

        lax.fori_loop(0, CHUNK // LANES, _group, 0)

        pltpu.sync_copy(rows_v, acc_sh.at[dst_v.at[c]], add=True)
        return carry

    lax.fori_loop(0, CHUNKS_PER_TILE, _chunk, 0)

    plsc.subcore_barrier()

    pltpu.sync_copy(
        acc_sh.at[pl.ds(base, ROWS_PER_TILE)],
        part_hbm.at[cid, pl.ds(base, ROWS_PER_TILE)],
    )


def kernel(x, W, edge_index, adj_vals):
    h = _matmul(x, W)
    pad = E_PAD - E
    dst = jnp.pad(edge_index[0], (0, pad)).reshape(NW, CHUNKS_PER_TILE, CHUNK)
    src = jnp.pad(edge_index[1], (0, pad)).reshape(NW, CHUNKS_PER_TILE, CHUNK)
    val = jnp.pad(adj_vals, (0, pad)).reshape(NW, CHUNKS_PER_TILE, CHUNK)
    part = _sc_edge_kernel(h, src, dst, val)
    return _combine(part[0, :N], part[1, :N])
